# speedup vs baseline: 4.7414x; 4.7414x over previous
import functools

import jax
import jax.numpy as jnp
from jax import lax
from jax.experimental import pallas as pl
from jax.experimental.pallas import tpu as pltpu
from jax.experimental.pallas import tpu_sc as plsc

_N = 10000
_D = 128
_E = 320000

_NC = 2
_NS = 16
_N_PAD = 10240
_ROWS_PER_TILE = _N_PAD // _NS
_E_PER_SC = _E // _NC
_E_PER_TILE = _E_PER_SC // _NS
_CHUNK = 80
_N_CHUNKS = _E_PER_TILE // _CHUNK

_BM = 400


def _segsum_sc(y, src, dst):
  mesh = plsc.VectorSubcoreMesh(
      core_axis_name="c", subcore_axis_name="s",
      num_cores=_NC, num_subcores=_NS)

  @functools.partial(
      pl.kernel,
      mesh=mesh,
      out_type=jax.ShapeDtypeStruct((_NC, _N_PAD, _D), jnp.float32),
      scratch_types=[
          pltpu.MemorySpace.VMEM_SHARED((_N_PAD, _D), jnp.float32),
          pltpu.MemorySpace.VMEM((_CHUNK,), jnp.int32),
          pltpu.MemorySpace.VMEM((_CHUNK,), jnp.int32),
          pltpu.MemorySpace.VMEM((_CHUNK, _D), jnp.float32),
          pltpu.SemaphoreType.DMA,
      ],
  )
  def k(y_hbm, src_hbm, dst_hbm, out_hbm, acc, src_v, dst_v, rows_v, sem):
    c = lax.axis_index("c")
    s = lax.axis_index("s")

    zero16 = jnp.zeros((16,), jnp.float32)

    def zrow(i, carry):
      for j in range(_D // 16):
        rows_v[i, pl.ds(j * 16, 16)] = zero16
      return carry

    lax.fori_loop(0, _CHUNK, zrow, 0)
    row0 = s * _ROWS_PER_TILE
    for j in range(_ROWS_PER_TILE // _CHUNK):
      pltpu.sync_copy(rows_v, acc.at[pl.ds(row0 + j * _CHUNK, _CHUNK)])
    plsc.subcore_barrier()

    ebase = c * _E_PER_SC + s * _E_PER_TILE

    def body(kk, carry):
      off = pl.multiple_of(ebase + kk * _CHUNK, 8)
      pltpu.sync_copy(src_hbm.at[pl.ds(off, _CHUNK)], src_v)
      pltpu.sync_copy(dst_hbm.at[pl.ds(off, _CHUNK)], dst_v)
      pltpu.async_copy(y_hbm.at[src_v], rows_v, sem).wait()
      pltpu.sync_copy(rows_v, acc.at[dst_v], add=True)
      return carry

    lax.fori_loop(0, _N_CHUNKS, body, 0)
    plsc.subcore_barrier()

    pltpu.sync_copy(acc.at[pl.ds(row0, _ROWS_PER_TILE)],
                    out_hbm.at[c, pl.ds(row0, _ROWS_PER_TILE)])

  return k(y, src, dst)


def _mm2_body(h_ref, ws_ref, wn_ref, b_ref, s_ref, y_ref):
  hb = h_ref[...]
  s_ref[...] = jnp.dot(hb, ws_ref[...],
                       preferred_element_type=jnp.float32) + b_ref[...]
  y_ref[...] = jnp.dot(hb, wn_ref[...], preferred_element_type=jnp.float32)


def _mm2(h, ws, wn, b):
  return pl.pallas_call(
      _mm2_body,
      grid=(_N // _BM,),
      in_specs=[
          pl.BlockSpec((_BM, _D), lambda i: (i, 0)),
          pl.BlockSpec((_D, _D), lambda i: (0, 0)),
          pl.BlockSpec((_D, _D), lambda i: (0, 0)),
          pl.BlockSpec((1, _D), lambda i: (0, 0)),
      ],
      out_specs=[pl.BlockSpec((_BM, _D), lambda i: (i, 0))] * 2,
      out_shape=[jax.ShapeDtypeStruct((_N, _D), jnp.float32)] * 2,
  )(h, ws, wn, b.reshape(1, _D))


def _combine_body(h_ref, s_ref, z_ref, o_ref):
  o_ref[...] = h_ref[...] + jax.nn.relu(
      s_ref[...] + z_ref[0] + z_ref[1])


def _combine(h, s, z):
  return pl.pallas_call(
      _combine_body,
      grid=(_N // _BM,),
      in_specs=[
          pl.BlockSpec((_BM, _D), lambda i: (i, 0)),
          pl.BlockSpec((_BM, _D), lambda i: (i, 0)),
          pl.BlockSpec((_NC, _BM, _D), lambda i: (0, i, 0)),
      ],
      out_specs=pl.BlockSpec((_BM, _D), lambda i: (i, 0)),
      out_shape=jax.ShapeDtypeStruct((_N, _D), jnp.float32),
  )(h, s, z)


def _final_body(s_ref, z_ref, o_ref):
  o_ref[...] = s_ref[...] + z_ref[0] + z_ref[1]


def _final(s, z):
  return pl.pallas_call(
      _final_body,
      grid=(_N // _BM,),
      in_specs=[
          pl.BlockSpec((_BM, _D), lambda i: (i, 0)),
          pl.BlockSpec((_NC, _BM, _D), lambda i: (0, i, 0)),
      ],
      out_specs=pl.BlockSpec((_BM, _D), lambda i: (i, 0)),
      out_shape=jax.ShapeDtypeStruct((_N, _D), jnp.float32),
  )(s, z)


@jax.jit
def kernel(x, edge_index, W0s, W0n, b0, W1s, W1n, b1, W2s, W2n, b2):
  src = edge_index[0]
  dst = edge_index[1]
  params = [(W0s, W0n, b0), (W1s, W1n, b1), (W2s, W2n, b2)]
  h = x
  out = None
  for i, (ws, wn, b) in enumerate(params):
    s, y = _mm2(h, ws, wn, b)
    z = _segsum_sc(y, src, dst)
    if i < 2:
      h = _combine(h, s, z)
    else:
      out = _final(s, z)
  return out


# speedup vs baseline: 9.6752x; 2.0406x over previous
import functools

import jax
import jax.numpy as jnp
from jax import lax
from jax.experimental import pallas as pl
from jax.experimental.pallas import tpu as pltpu
from jax.experimental.pallas import tpu_sc as plsc

_N = 10000
_D = 128
_E = 320000

_NC = 2
_NS = 16
_N_PAD = 10240
_ROWS_PER_TILE = _N_PAD // _NS
_E_PER_SC = _E // _NC
_E_PER_TILE = _E_PER_SC // _NS
_CHUNK = 80
_N_CHUNKS = _E_PER_TILE // _CHUNK
_BATCH = 25

_BM = 400


def _segsum_sc(y, src3, dst3):
  mesh = plsc.VectorSubcoreMesh(
      core_axis_name="c", subcore_axis_name="s",
      num_cores=_NC, num_subcores=_NS)

  @functools.partial(
      pl.kernel,
      mesh=mesh,
      out_type=jax.ShapeDtypeStruct((_NC, _N_PAD, _D), jnp.float32),
      scratch_types=[
          pltpu.MemorySpace.VMEM_SHARED((_N_PAD, _D), jnp.float32),
          pltpu.MemorySpace.VMEM((_BATCH, _CHUNK), jnp.int32),
          pltpu.MemorySpace.VMEM((_BATCH, _CHUNK), jnp.int32),
          pltpu.MemorySpace.VMEM((_CHUNK, _D), jnp.float32),
          pltpu.MemorySpace.VMEM((_CHUNK, _D), jnp.float32),
          pltpu.SemaphoreType.DMA,
          pltpu.SemaphoreType.DMA,
      ],
  )
  def k(y_hbm, src_hbm, dst_hbm, out_hbm, acc, src_v, dst_v,
        rows0, rows1, g0, g1):
    c = lax.axis_index("c")
    s = lax.axis_index("s")
    w = c * _NS + s

    zero16 = jnp.zeros((16,), jnp.float32)

    def zrow(i, carry):
      for j in range(_D // 16):
        rows0[i, pl.ds(j * 16, 16)] = zero16
      return carry

    lax.fori_loop(0, _CHUNK, zrow, 0)
    row0 = s * _ROWS_PER_TILE
    for j in range(_ROWS_PER_TILE // _CHUNK):
      pltpu.sync_copy(rows0, acc.at[pl.ds(row0 + j * _CHUNK, _CHUNK)])
    plsc.subcore_barrier()

    def batch(b, carry):
      pltpu.sync_copy(src_hbm.at[w, b], src_v)
      pltpu.sync_copy(dst_hbm.at[w, b], dst_v)
      pltpu.async_copy(y_hbm.at[src_v.at[0]], rows0, g0)

      def body(t, carry2):
        j0 = 2 * t
        pltpu.async_copy(y_hbm.at[src_v.at[j0 + 1]], rows1, g1)
        pltpu.make_async_copy(y_hbm.at[src_v.at[j0]], rows0, g0).wait()
        pltpu.sync_copy(rows0, acc.at[dst_v.at[j0]], add=True)
        pltpu.async_copy(y_hbm.at[src_v.at[j0 + 2]], rows0, g0)
        pltpu.make_async_copy(y_hbm.at[src_v.at[j0 + 1]], rows1, g1).wait()
        pltpu.sync_copy(rows1, acc.at[dst_v.at[j0 + 1]], add=True)
        return carry2

      lax.fori_loop(0, (_BATCH - 1) // 2, body, 0)
      pltpu.make_async_copy(y_hbm.at[src_v.at[_BATCH - 1]], rows0, g0).wait()
      pltpu.sync_copy(rows0, acc.at[dst_v.at[_BATCH - 1]], add=True)
      return carry

    lax.fori_loop(0, _N_CHUNKS // _BATCH, batch, 0)
    plsc.subcore_barrier()

    pltpu.sync_copy(acc.at[pl.ds(row0, _ROWS_PER_TILE)],
                    out_hbm.at[c, pl.ds(row0, _ROWS_PER_TILE)])

  return k(y, src3, dst3)


def _mm2_body(h_ref, ws_ref, wn_ref, b_ref, s_ref, y_ref):
  hb = h_ref[...]
  s_ref[...] = jnp.dot(hb, ws_ref[...],
                       preferred_element_type=jnp.float32) + b_ref[...]
  y_ref[...] = jnp.dot(hb, wn_ref[...], preferred_element_type=jnp.float32)


def _mm2(h, ws, wn, b):
  return pl.pallas_call(
      _mm2_body,
      grid=(_N // _BM,),
      in_specs=[
          pl.BlockSpec((_BM, _D), lambda i: (i, 0)),
          pl.BlockSpec((_D, _D), lambda i: (0, 0)),
          pl.BlockSpec((_D, _D), lambda i: (0, 0)),
          pl.BlockSpec((1, _D), lambda i: (0, 0)),
      ],
      out_specs=[pl.BlockSpec((_BM, _D), lambda i: (i, 0))] * 2,
      out_shape=[jax.ShapeDtypeStruct((_N, _D), jnp.float32)] * 2,
  )(h, ws, wn, b.reshape(1, _D))


def _combine_body(h_ref, s_ref, z_ref, o_ref):
  o_ref[...] = h_ref[...] + jax.nn.relu(
      s_ref[...] + z_ref[0] + z_ref[1])


def _combine(h, s, z):
  return pl.pallas_call(
      _combine_body,
      grid=(_N // _BM,),
      in_specs=[
          pl.BlockSpec((_BM, _D), lambda i: (i, 0)),
          pl.BlockSpec((_BM, _D), lambda i: (i, 0)),
          pl.BlockSpec((_NC, _BM, _D), lambda i: (0, i, 0)),
      ],
      out_specs=pl.BlockSpec((_BM, _D), lambda i: (i, 0)),
      out_shape=jax.ShapeDtypeStruct((_N, _D), jnp.float32),
  )(h, s, z)


def _final_body(s_ref, z_ref, o_ref):
  o_ref[...] = s_ref[...] + z_ref[0] + z_ref[1]


def _final(s, z):
  return pl.pallas_call(
      _final_body,
      grid=(_N // _BM,),
      in_specs=[
          pl.BlockSpec((_BM, _D), lambda i: (i, 0)),
          pl.BlockSpec((_NC, _BM, _D), lambda i: (0, i, 0)),
      ],
      out_specs=pl.BlockSpec((_BM, _D), lambda i: (i, 0)),
      out_shape=jax.ShapeDtypeStruct((_N, _D), jnp.float32),
  )(s, z)


@jax.jit
def kernel(x, edge_index, W0s, W0n, b0, W1s, W1n, b1, W2s, W2n, b2):
  src = edge_index[0].reshape(_NC * _NS, _N_CHUNKS // _BATCH, _BATCH, _CHUNK)
  dst = edge_index[1].reshape(_NC * _NS, _N_CHUNKS // _BATCH, _BATCH, _CHUNK)
  params = [(W0s, W0n, b0), (W1s, W1n, b1), (W2s, W2n, b2)]
  h = x
  out = None
  for i, (ws, wn, b) in enumerate(params):
    s, y = _mm2(h, ws, wn, b)
    z = _segsum_sc(y, src, dst)
    if i < 2:
      h = _combine(h, s, z)
    else:
      out = _final(s, z)
  return out


# speedup vs baseline: 12.9657x; 1.3401x over previous
import functools

import jax
import jax.numpy as jnp
from jax import lax
from jax.experimental import pallas as pl
from jax.experimental.pallas import tpu as pltpu
from jax.experimental.pallas import tpu_sc as plsc

_N = 10000
_D = 128
_E = 320000

_NC = 2
_NS = 16
_N_PAD = 10240
_ROWS_PER_TILE = _N_PAD // _NS
_E_PER_SC = _E // _NC
_E_PER_TILE = _E_PER_SC // _NS
_CHUNK = 80
_N_CHUNKS = _E_PER_TILE // _CHUNK
_BATCH = 25

_BM = 2000


def _segsum_sc(y, ei5):
  mesh = plsc.VectorSubcoreMesh(
      core_axis_name="c", subcore_axis_name="s",
      num_cores=_NC, num_subcores=_NS)

  @functools.partial(
      pl.kernel,
      mesh=mesh,
      out_type=jax.ShapeDtypeStruct((_NC, _N_PAD, _D), jnp.float32),
      scratch_types=[
          pltpu.MemorySpace.VMEM_SHARED((_N_PAD, _D), jnp.float32),
          pltpu.MemorySpace.VMEM((_BATCH, _CHUNK), jnp.int32),
          pltpu.MemorySpace.VMEM((_BATCH, _CHUNK), jnp.int32),
          pltpu.MemorySpace.VMEM((_CHUNK, _D), jnp.float32),
          pltpu.MemorySpace.VMEM((_CHUNK, _D), jnp.float32),
          pltpu.MemorySpace.VMEM((_CHUNK, _D), jnp.float32),
          [pltpu.SemaphoreType.DMA] * 3,
          [pltpu.SemaphoreType.DMA] * 3,
      ],
  )
  def k(y_hbm, ei_hbm, out_hbm, acc, src_v, dst_v,
        rows0, rows1, rows2, gg, ss):
    c = lax.axis_index("c")
    s = lax.axis_index("s")
    w = c * _NS + s

    pltpu.sync_copy(ei_hbm.at[0, w, 0], src_v)
    pltpu.sync_copy(ei_hbm.at[1, w, 0], dst_v)

    zero16 = jnp.zeros((16,), jnp.float32)

    def zrow(i, carry):
      for j in range(_D // 16):
        rows1[i, pl.ds(j * 16, 16)] = zero16
      return carry

    lax.fori_loop(0, _CHUNK, zrow, 0)
    row0 = s * _ROWS_PER_TILE

    pltpu.async_copy(y_hbm.at[src_v.at[0]], rows0, gg[0])
    for j in range(_ROWS_PER_TILE // _CHUNK):
      pltpu.sync_copy(rows1, acc.at[pl.ds(row0 + j * _CHUNK, _CHUNK)])
    plsc.subcore_barrier()

    rows = [rows0, rows1, rows2]

    def _gather(j, i):
      pltpu.async_copy(y_hbm.at[src_v.at[j]], rows[i], gg[i])

    def _gwait(j, i):
      pltpu.make_async_copy(y_hbm.at[src_v.at[j]], rows[i], gg[i]).wait()

    def _scat(j, i):
      pltpu.async_copy(rows[i], acc.at[dst_v.at[j]], ss[i], add=True)

    def _swait(j, i):
      pltpu.make_async_copy(rows[i], acc.at[dst_v.at[j]], ss[i]).wait()

    def batch(b, carry):
      first = b == 0 if isinstance(b, int) else False
      if not first:
        pltpu.sync_copy(ei_hbm.at[0, w, b], src_v)
        pltpu.sync_copy(ei_hbm.at[1, w, b], dst_v)
        _gather(0, 0)
      _gather(1, 1)
      _gwait(0, 0)
      _scat(0, 0)
      _gather(2, 2)

      def body(t, carry2):
        for k, i in enumerate((1, 2, 0)):
          j = 3 * t + 1 + k
          _gwait(j, i)
          _scat(j, i)
          _swait(j - 1, k)
          _gather(j + 2, k)
        return carry2

      lax.fori_loop(0, (_BATCH - 4) // 3, body, 0)
      je = _BATCH - 3
      i0, i1, i2 = je % 3, (je + 1) % 3, (je + 2) % 3
      _gwait(je, i0)
      _scat(je, i0)
      _swait(je - 1, (je - 1) % 3)
      _gather(je + 2, (je - 1) % 3)
      _gwait(je + 1, i1)
      _scat(je + 1, i1)
      _gwait(je + 2, i2)
      _scat(je + 2, i2)
      _swait(je, i0)
      _swait(je + 1, i1)
      _swait(je + 2, i2)
      return carry

    batch(0, 0)
    lax.fori_loop(1, _N_CHUNKS // _BATCH, batch, 0)
    plsc.subcore_barrier()

    pltpu.sync_copy(acc.at[pl.ds(row0, _ROWS_PER_TILE)],
                    out_hbm.at[c, pl.ds(row0, _ROWS_PER_TILE)])

  return k(y, ei5)


def _mm2_body(h_ref, ws_ref, wn_ref, b_ref, s_ref, y_ref):
  hb = h_ref[...]
  s_ref[...] = jnp.dot(hb, ws_ref[...],
                       preferred_element_type=jnp.float32) + b_ref[...]
  y_ref[...] = jnp.dot(hb, wn_ref[...], preferred_element_type=jnp.float32)


def _mm2(h, ws, wn, b):
  return pl.pallas_call(
      _mm2_body,
      grid=(_N // _BM,),
      in_specs=[
          pl.BlockSpec((_BM, _D), lambda i: (i, 0)),
          pl.BlockSpec((_D, _D), lambda i: (0, 0)),
          pl.BlockSpec((_D, _D), lambda i: (0, 0)),
          pl.BlockSpec((1, _D), lambda i: (0, 0)),
      ],
      out_specs=[pl.BlockSpec((_BM, _D), lambda i: (i, 0))] * 2,
      out_shape=[jax.ShapeDtypeStruct((_N, _D), jnp.float32)] * 2,
  )(h, ws, wn, b.reshape(1, _D))


def _cmb_mm2_body(h_ref, sp_ref, z_ref, ws_ref, wn_ref, b_ref,
                  h_out_ref, s_ref, y_ref):
  hn = h_ref[...] + jax.nn.relu(sp_ref[...] + z_ref[0] + z_ref[1])
  if h_out_ref is not None:
    h_out_ref[...] = hn
  s_ref[...] = jnp.dot(hn, ws_ref[...],
                       preferred_element_type=jnp.float32) + b_ref[...]
  y_ref[...] = jnp.dot(hn, wn_ref[...], preferred_element_type=jnp.float32)


def _cmb_mm2(h, sp, z, ws, wn, b, keep_h):
  n_out = 3 if keep_h else 2
  if keep_h:
    body = _cmb_mm2_body
  else:
    def body(h_ref, sp_ref, z_ref, ws_ref, wn_ref, b_ref, s_ref, y_ref):
      _cmb_mm2_body(h_ref, sp_ref, z_ref, ws_ref, wn_ref, b_ref,
                    None, s_ref, y_ref)

  return pl.pallas_call(
      body,
      grid=(_N // _BM,),
      in_specs=[
          pl.BlockSpec((_BM, _D), lambda i: (i, 0)),
          pl.BlockSpec((_BM, _D), lambda i: (i, 0)),
          pl.BlockSpec((_NC, _BM, _D), lambda i: (0, i, 0)),
          pl.BlockSpec((_D, _D), lambda i: (0, 0)),
          pl.BlockSpec((_D, _D), lambda i: (0, 0)),
          pl.BlockSpec((1, _D), lambda i: (0, 0)),
      ],
      out_specs=[pl.BlockSpec((_BM, _D), lambda i: (i, 0))] * n_out,
      out_shape=[jax.ShapeDtypeStruct((_N, _D), jnp.float32)] * n_out,
  )(h, sp, z, ws, wn, b.reshape(1, _D))


def _final_body(s_ref, z_ref, o_ref):
  o_ref[...] = s_ref[...] + z_ref[0] + z_ref[1]


def _final(s, z):
  return pl.pallas_call(
      _final_body,
      grid=(_N // _BM,),
      in_specs=[
          pl.BlockSpec((_BM, _D), lambda i: (i, 0)),
          pl.BlockSpec((_NC, _BM, _D), lambda i: (0, i, 0)),
      ],
      out_specs=pl.BlockSpec((_BM, _D), lambda i: (i, 0)),
      out_shape=jax.ShapeDtypeStruct((_N, _D), jnp.float32),
  )(s, z)


@jax.jit
def kernel(x, edge_index, W0s, W0n, b0, W1s, W1n, b1, W2s, W2n, b2):
  ei5 = edge_index.reshape(2, _NC * _NS, _N_CHUNKS // _BATCH, _BATCH, _CHUNK)
  s, y = _mm2(x, W0s, W0n, b0)
  z = _segsum_sc(y, ei5)
  h, s, y = _cmb_mm2(x, s, z, W1s, W1n, b1, keep_h=True)
  z = _segsum_sc(y, ei5)
  s, y = _cmb_mm2(h, s, z, W2s, W2n, b2, keep_h=False)
  z = _segsum_sc(y, ei5)
  return _final(s, z)
